# baseline (speedup 1.0000x reference)
import functools

import jax
import jax.numpy as jnp
from jax import lax
from jax.experimental import pallas as pl
from jax.experimental.pallas import tpu as pltpu
from jax.experimental.pallas import tpu_sc as plsc

CHUNK = 128
NBUF = 8
H = 2


@functools.lru_cache(maxsize=None)
def _build_gather(nchunks: int, nrows: int, d: int):
    info = plsc.get_sparse_core_info()
    nc, ns = info.num_cores, info.num_subcores
    nw = nc * ns
    assert nchunks % (nw * NBUF) == 0
    cpw = nchunks // nw
    ngrp = cpw // NBUF

    mesh = plsc.VectorSubcoreMesh(core_axis_name="c", subcore_axis_name="s")

    @functools.partial(
        pl.kernel,
        mesh=mesh,
        compiler_params=pltpu.CompilerParams(use_tc_tiling_on_sc=False),
        out_type=jax.ShapeDtypeStruct((nchunks * CHUNK, d), jnp.float32),
        scratch_types=[
            pltpu.VMEM((cpw, CHUNK), jnp.int32),
            pltpu.VMEM((NBUF, CHUNK, d), jnp.float32),
        ]
        + [pltpu.SemaphoreType.DMA] * NBUF
        + [pltpu.SemaphoreType.DMA] * NBUF,
    )
    def emb(idx_hbm, table_hbm, out_hbm, idx_v, rows_v, *sems):
        gsem, osem = sems[:NBUF], sems[NBUF:]
        wid = lax.axis_index("s") * nc + lax.axis_index("c")
        base = wid * cpw

        def gather(c, b):
            return pltpu.make_async_copy(
                table_hbm.at[idx_v.at[c]], rows_v.at[b], gsem[b])

        def out_copy(s, b):
            return pltpu.make_async_copy(
                rows_v.at[b], out_hbm.at[pl.ds((base + s) * CHUNK, CHUNK)],
                osem[b])

        pltpu.sync_copy(idx_hbm.at[pl.ds(base, cpw)], idx_v)
        for c in range(NBUF - H):
            gather(c, c).start()

        def group(g, first, last):
            for b in range(NBUF):
                s = g * NBUF + b
                gather(s, b).wait()
                out_copy(s, b).start()
                if not (last and b >= H):
                    br = (b - H) % NBUF
                    if not (first and b < H):
                        out_copy(s - H, br).wait()
                    gather(s + NBUF - H, br).start()

        group(0, True, False)
        lax.fori_loop(
            1, ngrp - 1,
            lambda g, carry: (group(g, False, False), carry)[1], 0)
        group(ngrp - 1, False, True)
        for b in range(NBUF):
            out_copy((ngrp - 1) * NBUF + b, b).wait()

    return emb


def _tc_compact(table, ncodes, d):
    rblk = 8000
    nblk = ncodes // rblk

    def body(in_ref, out_ref):
        v3 = in_ref[...].reshape(rblk // 2, 2, d)
        out_ref[...] = jnp.concatenate([v3[:, 0, :], v3[:, 1, :]], axis=1)

    return pl.pallas_call(
        body,
        grid=(nblk,),
        in_specs=[pl.BlockSpec((rblk, d), lambda i: (i, 0))],
        out_specs=pl.BlockSpec((rblk * d // 128, 128), lambda i: (i, 0)),
        out_shape=jax.ShapeDtypeStruct((ncodes * d // 128, 128), jnp.float32),
    )(table)


def _tc_transpose(g2, batch, hist, d):
    tpr = 128 // d
    qn = hist // tpr
    bblk = 512
    nblk = batch // bblk

    def body(in_ref, out_ref):
        v = in_ref[...]
        vv = v.reshape(bblk, qn, 128)
        for q in range(qn):
            out_ref[pl.ds(q * 128, 128), :] = jnp.transpose(vv[:, q, :])

    return pl.pallas_call(
        body,
        grid=(nblk,),
        in_specs=[pl.BlockSpec((bblk * qn, 128), lambda i: (i, 0))],
        out_specs=pl.BlockSpec((hist * d, bblk), lambda i: (0, i)),
        out_shape=jax.ShapeDtypeStruct((hist * d, batch), jnp.float32),
    )(g2)


def kernel(x, rev_x, table):
    batch, hist = x.shape
    ncodes, d = table.shape
    n = batch * hist
    assert n % CHUNK == 0

    tab2 = _tc_compact(table, ncodes, d).reshape(ncodes, d)

    xf = x.astype(jnp.int32).reshape(n // CHUNK, CHUNK)
    rf = rev_x.astype(jnp.int32).reshape(n // CHUNK, CHUNK)

    gfn = _build_gather(n // CHUNK, ncodes, d)
    outs = []
    for idx in (xf, rf):
        g = gfn(idx, tab2)
        g2 = g.reshape(n * d // 128, 128)
        p2 = _tc_transpose(g2, batch, hist, d)
        outs.append(p2.reshape(hist, d, batch).transpose(2, 0, 1))
    return outs[0], outs[1]


# speedup vs baseline: 1.1058x; 1.1058x over previous
import functools

import jax
import jax.numpy as jnp
from jax import lax
from jax.experimental import pallas as pl
from jax.experimental.pallas import tpu as pltpu
from jax.experimental.pallas import tpu_sc as plsc

CHUNK = 128
NBUF = 8
H = 2


@functools.lru_cache(maxsize=None)
def _build_gather(nchunks: int, nrows: int, d: int):
    info = plsc.get_sparse_core_info()
    nc, ns = info.num_cores, info.num_subcores
    nw = nc * ns
    assert nchunks % (nw * NBUF) == 0
    cpw = nchunks // nw
    ngrp = cpw // NBUF

    mesh = plsc.VectorSubcoreMesh(core_axis_name="c", subcore_axis_name="s")

    @functools.partial(
        pl.kernel,
        mesh=mesh,
        compiler_params=pltpu.CompilerParams(use_tc_tiling_on_sc=False),
        out_type=jax.ShapeDtypeStruct((nchunks * CHUNK, d), jnp.float32),
        scratch_types=[
            pltpu.VMEM((cpw, CHUNK), jnp.int32),
            pltpu.VMEM((NBUF, CHUNK, d), jnp.float32),
        ]
        + [pltpu.SemaphoreType.DMA] * NBUF
        + [pltpu.SemaphoreType.DMA] * NBUF,
    )
    def emb(idx_hbm, table_hbm, out_hbm, idx_v, rows_v, *sems):
        gsem, osem = sems[:NBUF], sems[NBUF:]
        wid = lax.axis_index("s") * nc + lax.axis_index("c")
        base = wid * cpw

        def gather(c, b):
            return pltpu.make_async_copy(
                table_hbm.at[idx_v.at[c]], rows_v.at[b], gsem[b])

        def out_copy(s, b):
            return pltpu.make_async_copy(
                rows_v.at[b], out_hbm.at[pl.ds((base + s) * CHUNK, CHUNK)],
                osem[b])

        pltpu.sync_copy(idx_hbm.at[pl.ds(base, cpw)], idx_v)
        for c in range(NBUF - H):
            gather(c, c).start()

        def group(g, first, last):
            for b in range(NBUF):
                s = g * NBUF + b
                gather(s, b).wait()
                out_copy(s, b).start()
                if not (last and b >= H):
                    br = (b - H) % NBUF
                    if not (first and b < H):
                        out_copy(s - H, br).wait()
                    gather(s + NBUF - H, br).start()

        group(0, True, False)
        lax.fori_loop(
            1, ngrp - 1,
            lambda g, carry: (group(g, False, False), carry)[1], 0)
        group(ngrp - 1, False, True)
        for b in range(NBUF):
            out_copy((ngrp - 1) * NBUF + b, b).wait()

    return emb


def _tc_compact(table, ncodes, d):
    rblk = 8000
    nblk = ncodes // rblk

    def body(in_ref, out_ref):
        v3 = in_ref[...].reshape(rblk // 2, 2, d)
        out_ref[...] = jnp.concatenate([v3[:, 0, :], v3[:, 1, :]], axis=1)

    return pl.pallas_call(
        body,
        grid=(nblk,),
        in_specs=[pl.BlockSpec((rblk, d), lambda i: (i, 0))],
        out_specs=pl.BlockSpec((rblk * d // 128, 128), lambda i: (i, 0)),
        out_shape=jax.ShapeDtypeStruct((ncodes * d // 128, 128), jnp.float32),
    )(table)


def _tc_transpose(g2, batch, hist, d):
    tpr = 128 // d
    qn = hist // tpr
    bblk = 512
    nblk = batch // bblk

    def body(in_ref, out_ref):
        v = in_ref[...]
        vv = v.reshape(bblk, qn, 128)
        for q in range(qn):
            out_ref[pl.ds(q * 128, 128), :] = jnp.transpose(vv[:, q, :])

    return pl.pallas_call(
        body,
        grid=(nblk,),
        in_specs=[pl.BlockSpec((bblk * qn, 128), lambda i: (i, 0))],
        out_specs=pl.BlockSpec((hist * d, bblk), lambda i: (0, i)),
        out_shape=jax.ShapeDtypeStruct((hist * d, batch), jnp.float32),
    )(g2)


def kernel(x, rev_x, table):
    batch, hist = x.shape
    ncodes, d = table.shape
    n = batch * hist
    assert n % CHUNK == 0

    tab2 = table

    xf = x.astype(jnp.int32).reshape(n // CHUNK, CHUNK)
    rf = rev_x.astype(jnp.int32).reshape(n // CHUNK, CHUNK)

    gfn = _build_gather(n // CHUNK, ncodes, d)
    outs = []
    for idx in (xf, rf):
        g = gfn(idx, tab2)
        g2 = g.reshape(n * d // 128, 128)
        p2 = _tc_transpose(g2, batch, hist, d)
        outs.append(p2.reshape(hist, d, batch).transpose(2, 0, 1))
    return outs[0], outs[1]


# speedup vs baseline: 1.1665x; 1.0549x over previous
import functools

import jax
import jax.numpy as jnp
from jax import lax
from jax.experimental import pallas as pl
from jax.experimental.pallas import tpu as pltpu
from jax.experimental.pallas import tpu_sc as plsc

CHUNK = 128
NBUF = 8
H = 2


@functools.lru_cache(maxsize=None)
def _build_gather(nchunks: int, nrows: int, d: int):
    info = plsc.get_sparse_core_info()
    nc, ns = info.num_cores, info.num_subcores
    nw = nc * ns
    assert nchunks % (nw * NBUF) == 0
    cpw = nchunks // nw
    ngrp = cpw // NBUF

    mesh = plsc.VectorSubcoreMesh(core_axis_name="c", subcore_axis_name="s")

    @functools.partial(
        pl.kernel,
        mesh=mesh,
        compiler_params=pltpu.CompilerParams(use_tc_tiling_on_sc=False),
        out_type=jax.ShapeDtypeStruct((nchunks * CHUNK, d), jnp.float32),
        scratch_types=[
            pltpu.VMEM((cpw, CHUNK), jnp.int32),
            pltpu.VMEM((NBUF, CHUNK, d), jnp.float32),
        ]
        + [pltpu.SemaphoreType.DMA] * NBUF
        + [pltpu.SemaphoreType.DMA] * NBUF,
    )
    def emb(idx_hbm, table_hbm, out_hbm, idx_v, rows_v, *sems):
        gsem, osem = sems[:NBUF], sems[NBUF:]
        wid = lax.axis_index("s") * nc + lax.axis_index("c")
        base = wid * cpw

        def gather(c, b):
            return pltpu.make_async_copy(
                table_hbm.at[idx_v.at[c]], rows_v.at[b], gsem[b])

        def out_copy(s, b):
            return pltpu.make_async_copy(
                rows_v.at[b], out_hbm.at[pl.ds((base + s) * CHUNK, CHUNK)],
                osem[b])

        pltpu.sync_copy(idx_hbm.at[pl.ds(base, cpw)], idx_v)
        for c in range(NBUF - H):
            gather(c, c).start()

        def group(g, first, last):
            for b in range(NBUF):
                s = g * NBUF + b
                gather(s, b).wait()
                out_copy(s, b).start()
                if not (last and b >= H):
                    br = (b - H) % NBUF
                    if not (first and b < H):
                        out_copy(s - H, br).wait()
                    gather(s + NBUF - H, br).start()

        group(0, True, False)
        lax.fori_loop(
            1, ngrp - 1,
            lambda g, carry: (group(g, False, False), carry)[1], 0)
        group(ngrp - 1, False, True)
        for b in range(NBUF):
            out_copy((ngrp - 1) * NBUF + b, b).wait()

    return emb


def _tc_transpose(g2, batch, hist, d):
    tpr = 128 // d
    qn = hist // tpr
    bblk = 512
    nblk = batch // bblk

    def body(in_ref, out_ref):
        v = in_ref[...]
        vv = v.reshape(bblk, qn, 128)
        for q in range(qn):
            out_ref[pl.ds(q * 128, 128), :] = jnp.transpose(vv[:, q, :])

    return pl.pallas_call(
        body,
        grid=(nblk,),
        in_specs=[pl.BlockSpec((bblk * qn, 128), lambda i: (i, 0))],
        out_specs=pl.BlockSpec((hist * d, bblk), lambda i: (0, i)),
        out_shape=jax.ShapeDtypeStruct((hist * d, batch), jnp.float32),
    )(g2)


def kernel(x, rev_x, table):
    batch, hist = x.shape
    ncodes, d = table.shape
    n = batch * hist
    assert n % CHUNK == 0

    tabp = jnp.pad(table, ((0, 0), (0, 128 - d)))
    tab2 = tabp.reshape(2 * ncodes, d)

    xf = (x.astype(jnp.int32) * 2).reshape(n // CHUNK, CHUNK)
    rf = (rev_x.astype(jnp.int32) * 2).reshape(n // CHUNK, CHUNK)

    gfn = _build_gather(n // CHUNK, 2 * ncodes, d)
    outs = []
    for idx in (xf, rf):
        g = gfn(idx, tab2)
        g2 = g.reshape(n * d // 128, 128)
        p2 = _tc_transpose(g2, batch, hist, d)
        outs.append(p2.reshape(hist, d, batch).transpose(2, 0, 1))
    return outs[0], outs[1]
